# baseline (speedup 1.0000x reference)
import dataclasses
import functools

import jax
import jax.numpy as jnp
from jax import lax
from jax.experimental import pallas as pl
from jax.experimental.pallas import tpu as pltpu
from jax.experimental.pallas import tpu_sc as plsc

N = 10000
E = 320000
D = 128
NC = 2
NS = 16
NW = NC * NS
CHUNK = 80
CPW = 125
EPW = CPW * CHUNK
DEPTH = 4
NB = (CPW - 1) // DEPTH
DPACK = D // 2
STRIPE = 624
TAIL = N - NS * STRIPE
ZROWS = 48


def _sc_gather_scale_scatter(hp, ei_flat, attn):
    mesh = plsc.VectorSubcoreMesh(core_axis_name="c", subcore_axis_name="s")
    DP = D // 2

    scratch = []
    for _ in range(DEPTH):
        scratch += [
            pltpu.VMEM((CHUNK,), jnp.int32),
            pltpu.VMEM((CHUNK,), jnp.int32),
            pltpu.VMEM((CHUNK,), jnp.float32),
            pltpu.VMEM((CHUNK, DP), jnp.int32),
            pltpu.VMEM((CHUNK,), jnp.int32),
        ]
    scratch += [pltpu.VMEM((CHUNK, D), jnp.float32)] * 2
    scratch += [
        pltpu.VMEM((ZROWS, D), jnp.float32),
        pltpu.VMEM_SHARED((N, D), jnp.float32),
    ]
    scratch += [pltpu.SemaphoreType.DMA] * (3 * DEPTH)

    cp = pltpu.CompilerParams()
    if "needs_layout_passes" in pltpu.CompilerParams.__dataclass_fields__:
        cp = dataclasses.replace(cp, needs_layout_passes=False)
    if "use_tc_tiling_on_sc" in pltpu.CompilerParams.__dataclass_fields__:
        cp = dataclasses.replace(cp, use_tc_tiling_on_sc=False)

    @functools.partial(
        pl.kernel,
        mesh=mesh,
        out_type=jax.ShapeDtypeStruct((NC, N, D), jnp.float32),
        scratch_types=scratch,
        compiler_params=cp,
    )
    def k(h_hbm, ei_hbm, attn_hbm, out_hbm, *refs):
        src_b = [refs[5 * u + 0] for u in range(DEPTH)]
        dst_b = [refs[5 * u + 1] for u in range(DEPTH)]
        attn_b = [refs[5 * u + 2] for u in range(DEPTH)]
        rows_b = [refs[5 * u + 3] for u in range(DEPTH)]
        sdst_b = [refs[5 * u + 4] for u in range(DEPTH)]
        msg_b = [refs[5 * DEPTH], refs[5 * DEPTH + 1]]
        zero_v = refs[5 * DEPTH + 2]
        hn_sh = refs[5 * DEPTH + 3]
        nsem = refs[5 * DEPTH + 4: 5 * DEPTH + 4 + DEPTH]
        gsem = refs[5 * DEPTH + 4 + DEPTH: 5 * DEPTH + 4 + 2 * DEPTH]
        ssem = refs[5 * DEPTH + 4 + 2 * DEPTH: 5 * DEPTH + 4 + 3 * DEPTH]

        cid = lax.axis_index("c")
        sid = lax.axis_index("s")
        wid = sid * NC + cid
        base_e = wid * EPW
        last_eb = base_e + (CPW - 1) * CHUNK

        def idx_start(c, u):
            eb = jnp.minimum(base_e + c * CHUNK, last_eb)
            pltpu.async_copy(ei_hbm.at[pl.ds(eb, CHUNK)], src_b[u], nsem[u])
            pltpu.async_copy(ei_hbm.at[pl.ds(E + eb, CHUNK)], dst_b[u], nsem[u])
            pltpu.async_copy(attn_hbm.at[pl.ds(eb, CHUNK)], attn_b[u], nsem[u])

        def idx_wait(u):
            pltpu.make_async_copy(ei_hbm.at[pl.ds(0, CHUNK)], src_b[u], nsem[u]).wait()
            pltpu.make_async_copy(ei_hbm.at[pl.ds(0, CHUNK)], dst_b[u], nsem[u]).wait()
            pltpu.make_async_copy(attn_hbm.at[pl.ds(0, CHUNK)], attn_b[u], nsem[u]).wait()

        def gather_start(u):
            pltpu.async_copy(h_hbm.at[src_b[u]], rows_b[u], gsem[u])

        def gather_wait(u):
            pltpu.make_async_copy(h_hbm.at[src_b[u]], rows_b[u], gsem[u]).wait()

        def scatter_start(u):
            pltpu.async_copy(msg_b[u % 2], hn_sh.at[sdst_b[u]], ssem[u], add=True)

        def scatter_wait(u):
            pltpu.make_async_copy(msg_b[u % 2], hn_sh.at[sdst_b[u]], ssem[u]).wait()

        splat_dnums = lax.GatherDimensionNumbers(
            offset_dims=(), collapsed_slice_dims=(0,), start_index_map=(0,))
        himask = jnp.int32(-65536)

        def scale_rows(u):
            msg = msg_b[u % 2]
            for q in range(CHUNK // 16):
                av = attn_b[u][pl.ds(q * 16, 16)]
                for r in range(16):
                    e = q * 16 + r
                    sp = lax.gather(
                        av, jnp.full((16, 1), r, jnp.int32), splat_dnums,
                        slice_sizes=(1,),
                        mode=lax.GatherScatterMode.PROMISE_IN_BOUNDS)
                    for g in range(DP // 16):
                        w = rows_b[u][e, pl.ds(g * 16, 16)]
                        lo = plsc.bitcast(w << 16, jnp.float32)
                        hi = plsc.bitcast(w & himask, jnp.float32)
                        msg[e, pl.ds(g * 16, 16)] = lo * sp
                        msg[e, pl.ds(DP + g * 16, 16)] = hi * sp

        zeros16 = jnp.zeros((16,), jnp.float32)

        @pl.loop(0, ZROWS)
        def _(i):
            for j in range(D // 16):
                zero_v[i, pl.ds(j * 16, 16)] = zeros16

        for z in range(STRIPE // ZROWS):
            pltpu.sync_copy(
                zero_v, hn_sh.at[pl.ds(sid * STRIPE + z * ZROWS, ZROWS)])

        @pl.when(sid == NS - 1)
        def _():
            pltpu.sync_copy(zero_v.at[pl.ds(0, TAIL)],
                            hn_sh.at[pl.ds(NS * STRIPE, TAIL)])
        plsc.subcore_barrier()

        for u in range(DEPTH):
            idx_start(jnp.int32(u), u)
        for u in range(3):
            idx_wait(u)
            gather_start(u)

        @pl.loop(0, NB)
        def _(b):
            c0 = b * DEPTH
            for u in range(DEPTH):
                c = c0 + u
                gather_wait(u)
                scale_rows(u)
                for i in range(CHUNK // 16):
                    sl = pl.ds(i * 16, 16)
                    sdst_b[u][sl] = dst_b[u][sl]
                scatter_start(u)
                idx_start(c + DEPTH, u)
                u3 = (u + 3) % DEPTH
                idx_wait(u3)
                if u == 0:
                    @pl.when(b > 0)
                    def _():
                        scatter_wait(u3)
                else:
                    scatter_wait(u3)
                gather_start(u3)

        for s in (0, 1, 2):
            pltpu.make_async_copy(h_hbm.at[src_b[s]], rows_b[s], gsem[s]).wait()
        scatter_wait(3)
        idx_wait(3)

        scale_rows(0)
        for i in range(CHUNK // 16):
            sl = pl.ds(i * 16, 16)
            sdst_b[0][sl] = dst_b[0][sl]
        scatter_start(0)
        scatter_wait(0)

        plsc.subcore_barrier()
        pltpu.sync_copy(hn_sh.at[pl.ds(sid * STRIPE, STRIPE)],
                        out_hbm.at[cid, pl.ds(sid * STRIPE, STRIPE)])

        @pl.when(sid == NS - 1)
        def _():
            pltpu.sync_copy(hn_sh.at[pl.ds(NS * STRIPE, TAIL)],
                            out_hbm.at[cid, pl.ds(NS * STRIPE, TAIL)])

    return k(hp, ei_flat, attn)


_BLK = 1000


def _bi_interact(h_blk, hn, w1_ref, b1_ref, w2_ref, b2_ref):
    s = h_blk + hn
    p = h_blk * hn
    o1 = jnp.dot(s, w1_ref[...], preferred_element_type=jnp.float32,
                 precision=lax.Precision.HIGHEST) + b1_ref[...]
    o2 = jnp.dot(p, w2_ref[...], preferred_element_type=jnp.float32,
                 precision=lax.Precision.HIGHEST) + b2_ref[...]
    o = jnp.where(o1 >= 0, o1, 0.01 * o1) + jnp.where(o2 >= 0, o2, 0.01 * o2)
    nrm = jnp.sqrt(jnp.sum(o * o, axis=1, keepdims=True))
    return o, o / nrm


def _pack_rows(o):
    dp = o.shape[1] // 2
    au = lax.bitcast_convert_type(
        lax.convert_element_type(o[:, :dp], jnp.bfloat16), jnp.uint16
    ).astype(jnp.uint32)
    bu = lax.bitcast_convert_type(
        lax.convert_element_type(o[:, dp:], jnp.bfloat16), jnp.uint16
    ).astype(jnp.uint32)
    return lax.bitcast_convert_type((bu << 16) | au, jnp.int32)


def _tc_pack(x):
    def body(x_ref, p_ref):
        p_ref[...] = _pack_rows(x_ref[...])

    grid = (N // _BLK,)
    return pl.pallas_call(
        body,
        grid=grid,
        in_specs=[pl.BlockSpec((_BLK, D), lambda i: (i, 0))],
        out_specs=pl.BlockSpec((_BLK, DPACK), lambda i: (i, 0)),
        out_shape=jax.ShapeDtypeStruct((N, DPACK), jnp.int32),
    )(x)


def _tc_layer0(h, hnp, w1, b1, w2, b2):
    def body(h_ref, p_ref, w1_ref, b1_ref, w2_ref, b2_ref, o_ref, n_ref, pk_ref):
        hn = p_ref[0] + p_ref[1]
        o, n = _bi_interact(h_ref[...], hn, w1_ref, b1_ref, w2_ref, b2_ref)
        o_ref[...] = o
        n_ref[...] = n
        pk_ref[...] = _pack_rows(o)

    grid = (N // _BLK,)
    row_spec = pl.BlockSpec((_BLK, D), lambda i: (i, 0))
    p_spec = pl.BlockSpec((NC, _BLK, D), lambda i: (0, i, 0))
    w_spec = pl.BlockSpec((D, D), lambda i: (0, 0))
    b_spec = pl.BlockSpec((1, D), lambda i: (0, 0))
    return pl.pallas_call(
        body,
        grid=grid,
        in_specs=[row_spec, p_spec, w_spec, b_spec, w_spec, b_spec],
        out_specs=[row_spec, row_spec,
                   pl.BlockSpec((_BLK, DPACK), lambda i: (i, 0))],
        out_shape=[jax.ShapeDtypeStruct((N, D), jnp.float32),
                   jax.ShapeDtypeStruct((N, D), jnp.float32),
                   jax.ShapeDtypeStruct((N, DPACK), jnp.int32)],
    )(h, hnp, w1, b1.reshape(1, D), w2, b2.reshape(1, D))


def _tc_layer1(x, n1, h1, hnp, w1, b1, w2, b2):
    k_dim = w1.shape[1]
    width = 2 * D + k_dim

    def body(x_ref, n1_ref, h_ref, p_ref, w1_ref, b1_ref, w2_ref, b2_ref, o_ref):
        hn = p_ref[0] + p_ref[1]
        _, n2 = _bi_interact(h_ref[...], hn, w1_ref, b1_ref, w2_ref, b2_ref)
        o_ref[:, 0:D] = x_ref[...]
        o_ref[:, D:2 * D] = n1_ref[...]
        o_ref[:, 2 * D:width] = n2

    grid = (N // _BLK,)
    row_spec = pl.BlockSpec((_BLK, D), lambda i: (i, 0))
    p_spec = pl.BlockSpec((NC, _BLK, D), lambda i: (0, i, 0))
    w_spec = pl.BlockSpec((D, k_dim), lambda i: (0, 0))
    b_spec = pl.BlockSpec((1, k_dim), lambda i: (0, 0))
    return pl.pallas_call(
        body,
        grid=grid,
        in_specs=[row_spec, row_spec, row_spec, p_spec, w_spec, b_spec, w_spec,
                  b_spec],
        out_specs=pl.BlockSpec((_BLK, width), lambda i: (i, 0)),
        out_shape=jax.ShapeDtypeStruct((N, width), jnp.float32),
    )(x, n1, h1, hnp, w1, b1.reshape(1, k_dim), w2, b2.reshape(1, k_dim))


def kernel(x, edge_index, edge_attn, W1w0, W1b0, W2w0, W2b0, W1w1, W1b1, W2w1, W2b1):
    ei_flat = edge_index.reshape(2 * E)
    xp = _tc_pack(x)
    hn0p = _sc_gather_scale_scatter(xp, ei_flat, edge_attn)
    h1, n1, h1p = _tc_layer0(x, hn0p, W1w0, W1b0, W2w0, W2b0)

    hn1p = _sc_gather_scale_scatter(h1p, ei_flat, edge_attn)
    return _tc_layer1(x, n1, h1, hn1p, W1w1, W1b1, W2w1, W2b1)


# speedup vs baseline: 1.0991x; 1.0991x over previous
import dataclasses
import functools

import jax
import jax.numpy as jnp
from jax import lax
from jax.experimental import pallas as pl
from jax.experimental.pallas import tpu as pltpu
from jax.experimental.pallas import tpu_sc as plsc

N = 10000
E = 320000
D = 128
NC = 2
NS = 16
NW = NC * NS
CHUNK = 80
CPW = 125
EPW = CPW * CHUNK
DEPTH = 4
NB = (CPW - 1) // DEPTH
DPACK = D // 2
STRIPE = 624
TAIL = N - NS * STRIPE
ZROWS = 48


def _sc_gather_scale_scatter(hp, ei_flat, attn):
    mesh = plsc.VectorSubcoreMesh(core_axis_name="c", subcore_axis_name="s")
    DP = D // 2

    scratch = []
    for _ in range(DEPTH):
        scratch += [
            pltpu.VMEM((CHUNK,), jnp.int32),
            pltpu.VMEM((CHUNK,), jnp.int32),
            pltpu.VMEM((CHUNK,), jnp.float32),
            pltpu.VMEM((CHUNK, D), jnp.float32),
            pltpu.VMEM((CHUNK,), jnp.int32),
        ]
    scratch += [
        pltpu.VMEM((ZROWS, D), jnp.float32),
        pltpu.VMEM_SHARED((N, D), jnp.float32),
    ]
    scratch += [pltpu.SemaphoreType.DMA] * (3 * DEPTH)

    @functools.partial(
        pl.kernel,
        mesh=mesh,
        out_type=jax.ShapeDtypeStruct((NC, N, D), jnp.float32),
        scratch_types=scratch,
    )
    def k(h_hbm, ei_hbm, attn_hbm, out_hbm, *refs):
        src_b = [refs[5 * u + 0] for u in range(DEPTH)]
        dst_b = [refs[5 * u + 1] for u in range(DEPTH)]
        attn_b = [refs[5 * u + 2] for u in range(DEPTH)]
        rows_b = [refs[5 * u + 3] for u in range(DEPTH)]
        sdst_b = [refs[5 * u + 4] for u in range(DEPTH)]
        zero_v = refs[5 * DEPTH]
        hn_sh = refs[5 * DEPTH + 1]
        nsem = refs[5 * DEPTH + 2: 5 * DEPTH + 2 + DEPTH]
        gsem = refs[5 * DEPTH + 2 + DEPTH: 5 * DEPTH + 2 + 2 * DEPTH]
        ssem = refs[5 * DEPTH + 2 + 2 * DEPTH: 5 * DEPTH + 2 + 3 * DEPTH]

        cid = lax.axis_index("c")
        sid = lax.axis_index("s")
        wid = sid * NC + cid
        base_e = wid * EPW
        last_eb = base_e + (CPW - 1) * CHUNK

        def idx_start(c, u):
            eb = jnp.minimum(base_e + c * CHUNK, last_eb)
            pltpu.async_copy(ei_hbm.at[pl.ds(eb, CHUNK)], src_b[u], nsem[u])
            pltpu.async_copy(ei_hbm.at[pl.ds(E + eb, CHUNK)], dst_b[u], nsem[u])
            pltpu.async_copy(attn_hbm.at[pl.ds(eb, CHUNK)], attn_b[u], nsem[u])

        def idx_wait(u):
            pltpu.make_async_copy(ei_hbm.at[pl.ds(0, CHUNK)], src_b[u], nsem[u]).wait()
            pltpu.make_async_copy(ei_hbm.at[pl.ds(0, CHUNK)], dst_b[u], nsem[u]).wait()
            pltpu.make_async_copy(attn_hbm.at[pl.ds(0, CHUNK)], attn_b[u], nsem[u]).wait()

        def gather_start(u):
            pltpu.async_copy(h_hbm.at[src_b[u]], rows_b[u], gsem[u])

        def gather_wait(u):
            pltpu.make_async_copy(h_hbm.at[src_b[u]], rows_b[u], gsem[u]).wait()

        def scatter_start(u):
            pltpu.async_copy(rows_b[u], hn_sh.at[sdst_b[u]], ssem[u], add=True)

        def scatter_wait(u):
            pltpu.make_async_copy(rows_b[u], hn_sh.at[sdst_b[u]], ssem[u]).wait()

        splat_dnums = lax.GatherDimensionNumbers(
            offset_dims=(), collapsed_slice_dims=(0,), start_index_map=(0,))

        def scale_rows(u):
            for q in range(CHUNK // 16):
                av = attn_b[u][pl.ds(q * 16, 16)]
                for r in range(16):
                    e = q * 16 + r
                    sp = lax.gather(
                        av, jnp.full((16, 1), r, jnp.int32), splat_dnums,
                        slice_sizes=(1,),
                        mode=lax.GatherScatterMode.PROMISE_IN_BOUNDS)
                    for j in range(D // 16):
                        sl = pl.ds(j * 16, 16)
                        rows_b[u][e, sl] = rows_b[u][e, sl] * sp

        for u in range(DEPTH):
            idx_start(jnp.int32(u), u)

        zeros16 = jnp.zeros((16,), jnp.float32)

        @pl.loop(0, ZROWS)
        def _(i):
            for j in range(D // 16):
                zero_v[i, pl.ds(j * 16, 16)] = zeros16

        for u in range(3):
            idx_wait(u)
            gather_start(u)

        for z in range(STRIPE // ZROWS):
            pltpu.sync_copy(
                zero_v, hn_sh.at[pl.ds(sid * STRIPE + z * ZROWS, ZROWS)])

        @pl.when(sid == NS - 1)
        def _():
            pltpu.sync_copy(zero_v.at[pl.ds(0, TAIL)],
                            hn_sh.at[pl.ds(NS * STRIPE, TAIL)])
        plsc.subcore_barrier()


        @pl.loop(0, NB)
        def _(b):
            c0 = b * DEPTH
            for u in range(DEPTH):
                c = c0 + u
                gather_wait(u)
                scale_rows(u)
                for i in range(CHUNK // 16):
                    sl = pl.ds(i * 16, 16)
                    sdst_b[u][sl] = dst_b[u][sl]
                scatter_start(u)
                idx_start(c + DEPTH, u)
                u3 = (u + 3) % DEPTH
                idx_wait(u3)
                if u == 0:
                    @pl.when(b > 0)
                    def _():
                        scatter_wait(u3)
                else:
                    scatter_wait(u3)
                gather_start(u3)

        for s in (0, 1, 2):
            pltpu.make_async_copy(h_hbm.at[src_b[s]], rows_b[s], gsem[s]).wait()
        scatter_wait(3)
        idx_wait(3)

        scale_rows(0)
        for i in range(CHUNK // 16):
            sl = pl.ds(i * 16, 16)
            sdst_b[0][sl] = dst_b[0][sl]
        scatter_start(0)
        scatter_wait(0)

        plsc.subcore_barrier()
        pltpu.sync_copy(hn_sh.at[pl.ds(sid * STRIPE, STRIPE)],
                        out_hbm.at[cid, pl.ds(sid * STRIPE, STRIPE)])

        @pl.when(sid == NS - 1)
        def _():
            pltpu.sync_copy(hn_sh.at[pl.ds(NS * STRIPE, TAIL)],
                            out_hbm.at[cid, pl.ds(NS * STRIPE, TAIL)])

    return k(hp, ei_flat, attn)


_BLK = 1000


def _bi_interact(h_blk, hn, w1_ref, b1_ref, w2_ref, b2_ref):
    s = h_blk + hn
    p = h_blk * hn
    o1 = jnp.dot(s, w1_ref[...], preferred_element_type=jnp.float32,
                 precision=lax.Precision.HIGHEST) + b1_ref[...]
    o2 = jnp.dot(p, w2_ref[...], preferred_element_type=jnp.float32,
                 precision=lax.Precision.HIGHEST) + b2_ref[...]
    o = jnp.where(o1 >= 0, o1, 0.01 * o1) + jnp.where(o2 >= 0, o2, 0.01 * o2)
    nrm = jnp.sqrt(jnp.sum(o * o, axis=1, keepdims=True))
    return o, o / nrm


def _tc_layer0(h, hnp, w1, b1, w2, b2):
    def body(h_ref, p_ref, w1_ref, b1_ref, w2_ref, b2_ref, o_ref, n_ref):
        hn = p_ref[0] + p_ref[1]
        o, n = _bi_interact(h_ref[...], hn, w1_ref, b1_ref, w2_ref, b2_ref)
        o_ref[...] = o
        n_ref[...] = n

    grid = (N // _BLK,)
    row_spec = pl.BlockSpec((_BLK, D), lambda i: (i, 0))
    p_spec = pl.BlockSpec((NC, _BLK, D), lambda i: (0, i, 0))
    w_spec = pl.BlockSpec((D, D), lambda i: (0, 0))
    b_spec = pl.BlockSpec((1, D), lambda i: (0, 0))
    return pl.pallas_call(
        body,
        grid=grid,
        in_specs=[row_spec, p_spec, w_spec, b_spec, w_spec, b_spec],
        out_specs=[row_spec, row_spec],
        out_shape=[jax.ShapeDtypeStruct((N, D), jnp.float32),
                   jax.ShapeDtypeStruct((N, D), jnp.float32)],
    )(h, hnp, w1, b1.reshape(1, D), w2, b2.reshape(1, D))


def _tc_layer1(x, n1, h1, hnp, w1, b1, w2, b2):
    k_dim = w1.shape[1]
    width = 2 * D + k_dim

    def body(x_ref, n1_ref, h_ref, p_ref, w1_ref, b1_ref, w2_ref, b2_ref, o_ref):
        hn = p_ref[0] + p_ref[1]
        _, n2 = _bi_interact(h_ref[...], hn, w1_ref, b1_ref, w2_ref, b2_ref)
        o_ref[:, 0:D] = x_ref[...]
        o_ref[:, D:2 * D] = n1_ref[...]
        o_ref[:, 2 * D:width] = n2

    grid = (N // _BLK,)
    row_spec = pl.BlockSpec((_BLK, D), lambda i: (i, 0))
    p_spec = pl.BlockSpec((NC, _BLK, D), lambda i: (0, i, 0))
    w_spec = pl.BlockSpec((D, k_dim), lambda i: (0, 0))
    b_spec = pl.BlockSpec((1, k_dim), lambda i: (0, 0))
    return pl.pallas_call(
        body,
        grid=grid,
        in_specs=[row_spec, row_spec, row_spec, p_spec, w_spec, b_spec, w_spec,
                  b_spec],
        out_specs=pl.BlockSpec((_BLK, width), lambda i: (i, 0)),
        out_shape=jax.ShapeDtypeStruct((N, width), jnp.float32),
    )(x, n1, h1, hnp, w1, b1.reshape(1, k_dim), w2, b2.reshape(1, k_dim))


def kernel(x, edge_index, edge_attn, W1w0, W1b0, W2w0, W2b0, W1w1, W1b1, W2w1, W2b1):
    ei_flat = edge_index.reshape(2 * E)
    hn0p = _sc_gather_scale_scatter(x, ei_flat, edge_attn)
    h1, n1 = _tc_layer0(x, hn0p, W1w0, W1b0, W2w0, W2b0)

    hn1p = _sc_gather_scale_scatter(h1, ei_flat, edge_attn)
    return _tc_layer1(x, n1, h1, hn1p, W1w1, W1b1, W2w1, W2b1)
